# Initial kernel scaffold; baseline (speedup 1.0000x reference)
import jax
import jax.numpy as jnp
from jax.experimental import pallas as pl


def kernel(inFeatures, W1, b1, W2, b2):
    raise NotImplementedError("write your pallas kernel here")


# speedup vs baseline: 6.5340x; 6.5340x over previous
import numpy as np
import jax
import jax.numpy as jnp
from jax.experimental import pallas as pl
from jax.experimental.pallas import tpu as pltpu

_H = 35
_W = 35
_N = _H * _W
_NPAD = 1280
_CIN = 512
_CHID = 128
_COUT = 512


def _stencil_coeffs():
    rows = np.arange(_N) // _W
    cols = np.arange(_N) % _W
    deg = 1.0 + (cols > 0) + (cols < _W - 1) + (rows > 0) + (rows < _H - 1)
    u = deg ** -0.5
    c0 = u * u
    cL = np.where(cols > 0, u * np.roll(u, 1), 0.0)
    cR = np.where(cols < _W - 1, u * np.roll(u, -1), 0.0)
    cU = np.where(rows > 0, u * np.roll(u, _W), 0.0)
    cD = np.where(rows < _H - 1, u * np.roll(u, -_W), 0.0)
    r = c0 + cL + cR + cU + cD
    coef = np.zeros((8, _NPAD), np.float32)
    for i, v in enumerate((c0, cL, cR, cU, cD)):
        coef[i, :_N] = v
    rcol = np.zeros((_N, 1), np.float32)
    rcol[:, 0] = r
    return coef, rcol


_COEF_NP, _RCOL_NP = _stencil_coeffs()


def _gcn_body(x_ref, w1_ref, w2_ref, b1_ref, b2_ref, coef_ref, rcol_ref, o_ref):
    x = x_ref[0]
    z = jax.lax.dot_general(w1_ref[...], x, (((0,), (0,)), ((), ())),
                            preferred_element_type=jnp.float32)
    z = jnp.concatenate([z, jnp.zeros((_CHID, _NPAD - _N), jnp.float32)], axis=1)

    def prop(t):
        return (coef_ref[0:1, :] * t
                + coef_ref[1:2, :] * pltpu.roll(t, 1, 1)
                + coef_ref[2:3, :] * pltpu.roll(t, _NPAD - 1, 1)
                + coef_ref[3:4, :] * pltpu.roll(t, _W, 1)
                + coef_ref[4:5, :] * pltpu.roll(t, _NPAD - _W, 1))

    z2 = prop(prop(z))[:, :_N]
    y = jax.lax.dot_general(z2, w2_ref[...], (((0,), (0,)), ((), ())),
                            preferred_element_type=jnp.float32)
    bvec = jax.lax.dot_general(b1_ref[...], w2_ref[...], (((1,), (0,)), ((), ())),
                               preferred_element_type=jnp.float32)
    o_ref[0] = y + rcol_ref[...] * bvec + b2_ref[...]


def kernel(inFeatures, W1, b1, W2, b2):
    B = inFeatures.shape[0]
    xf = inFeatures.reshape(B, _CIN, _N)
    out = pl.pallas_call(
        _gcn_body,
        grid=(B,),
        in_specs=[
            pl.BlockSpec((1, _CIN, _N), lambda b: (b, 0, 0)),
            pl.BlockSpec((_CIN, _CHID), lambda b: (0, 0)),
            pl.BlockSpec((_CHID, _COUT), lambda b: (0, 0)),
            pl.BlockSpec((1, _CHID), lambda b: (0, 0)),
            pl.BlockSpec((1, _COUT), lambda b: (0, 0)),
            pl.BlockSpec((8, _NPAD), lambda b: (0, 0)),
            pl.BlockSpec((_N, 1), lambda b: (0, 0)),
        ],
        out_specs=pl.BlockSpec((1, _N, _COUT), lambda b: (b, 0, 0)),
        out_shape=jax.ShapeDtypeStruct((B, _N, _COUT), jnp.float32),
        compiler_params=pltpu.CompilerParams(dimension_semantics=("parallel",)),
    )(xf, W1, W2,
      b1.reshape(1, _CHID), b2.reshape(1, _COUT),
      jnp.asarray(_COEF_NP), jnp.asarray(_RCOL_NP))
    return out.reshape(B, _COUT, _H, _W)


# speedup vs baseline: 6.7373x; 1.0311x over previous
import numpy as np
import jax
import jax.numpy as jnp
from jax.experimental import pallas as pl
from jax.experimental.pallas import tpu as pltpu

_H = 35
_W = 35
_N = _H * _W
_B = 16
_CIN = 512
_CHID = 128
_COUT = 512
_R = _N * _B
_BLK = 2800
_HALO = 560
_NSTEP = _R // _BLK


def _stencil_coeffs():
    ii, jj = np.meshgrid(np.arange(_H), np.arange(_W), indexing="ij")
    deg = 1.0 + (jj > 0) + (jj < _W - 1) + (ii > 0) + (ii < _H - 1)
    u = deg ** -0.5
    c0 = u * u
    cL = np.where(jj > 0, u * np.roll(u, 1, axis=1), 0.0)
    cR = np.where(jj < _W - 1, u * np.roll(u, -1, axis=1), 0.0)
    cU = np.where(ii > 0, u * np.roll(u, 1, axis=0), 0.0)
    cD = np.where(ii < _H - 1, u * np.roll(u, -1, axis=0), 0.0)
    r = c0 + cL + cR + cU + cD
    coef = np.zeros((_R, 8), np.float32)
    for k, v in enumerate((c0, cL, cR, cU, cD)):
        coef[:, k] = np.repeat(v.reshape(-1), _B)
    coef[:, 5] = 1.0
    coef[:, 6] = np.repeat(r.reshape(-1), _B)
    return coef


_COEF_NP = _stencil_coeffs()


def _gcn_body(x_ref, xh1_ref, xh2_ref, xh3_ref, xh4_ref,
              c_ref, ch1_ref, ch2_ref, ch3_ref, ch4_ref,
              w1_ref, w2_ref, b1_ref, b2_ref, o_ref):
    x = jnp.concatenate([xh1_ref[...], xh2_ref[...], x_ref[...],
                         xh3_ref[...], xh4_ref[...]], axis=0)
    c = jnp.concatenate([ch1_ref[...], ch2_ref[...], c_ref[...],
                         ch3_ref[...], ch4_ref[...]], axis=0)
    z = jax.lax.dot_general(x, w1_ref[...], (((1,), (0,)), ((), ())),
                            preferred_element_type=jnp.float32)

    def prop(t):
        return (c[:, 0:1] * t
                + c[:, 1:2] * pltpu.roll(t, _B, 0)
                + c[:, 2:3] * pltpu.roll(t, t.shape[0] - _B, 0)
                + c[:, 3:4] * pltpu.roll(t, _HALO, 0)
                + c[:, 4:5] * pltpu.roll(t, t.shape[0] - _HALO, 0))

    z2 = prop(prop(z))[2 * _HALO: 2 * _HALO + _BLK]
    lhs = jnp.concatenate([z2, c_ref[:, 5:7]], axis=1)
    bvec = jax.lax.dot_general(b1_ref[...], w2_ref[...], (((1,), (0,)), ((), ())),
                               preferred_element_type=jnp.float32)
    w2a = jnp.concatenate([w2_ref[...], b2_ref[...], bvec], axis=0)
    o_ref[...] = jax.lax.dot_general(lhs, w2a, (((1,), (0,)), ((), ())),
                                     preferred_element_type=jnp.float32)


def _halo_specs(ncols):
    u = _BLK // _HALO
    return [
        pl.BlockSpec((_BLK, ncols), lambda i: (i, 0)),
        pl.BlockSpec((_HALO, ncols), lambda i: (jnp.maximum(i * u - 2, 0), 0)),
        pl.BlockSpec((_HALO, ncols), lambda i: (jnp.maximum(i * u - 1, 0), 0)),
        pl.BlockSpec((_HALO, ncols), lambda i: (jnp.minimum(i * u + u, _R // _HALO - 1), 0)),
        pl.BlockSpec((_HALO, ncols), lambda i: (jnp.minimum(i * u + u + 1, _R // _HALO - 1), 0)),
    ]


def kernel(inFeatures, W1, b1, W2, b2):
    xin = inFeatures.transpose(2, 3, 0, 1).reshape(_R, _CIN)
    y = pl.pallas_call(
        _gcn_body,
        grid=(_NSTEP,),
        in_specs=(
            _halo_specs(_CIN)
            + _halo_specs(8)
            + [
                pl.BlockSpec((_CIN, _CHID), lambda i: (0, 0)),
                pl.BlockSpec((_CHID, _COUT), lambda i: (0, 0)),
                pl.BlockSpec((1, _CHID), lambda i: (0, 0)),
                pl.BlockSpec((1, _COUT), lambda i: (0, 0)),
            ]
        ),
        out_specs=pl.BlockSpec((_BLK, _COUT), lambda i: (i, 0)),
        out_shape=jax.ShapeDtypeStruct((_R, _COUT), jnp.float32),
        compiler_params=pltpu.CompilerParams(dimension_semantics=("parallel",)),
    )(xin, xin, xin, xin, xin,
      jnp.asarray(_COEF_NP), jnp.asarray(_COEF_NP), jnp.asarray(_COEF_NP),
      jnp.asarray(_COEF_NP), jnp.asarray(_COEF_NP),
      W1, W2, b1.reshape(1, _CHID), b2.reshape(1, _COUT))
    return y.reshape(_N, _B, _COUT).transpose(1, 0, 2).reshape(_B, _COUT, _H, _W)
